# baseline (speedup 1.0000x reference)
import functools

import jax
import jax.numpy as jnp
from jax import lax
from jax.experimental import pallas as pl
from jax.experimental.pallas import tpu as pltpu
from jax.experimental.pallas import tpu_sc as plsc

B, T, D = 32, 256, 1024
BT = B * T
NSLOT = 4
CH = 64
GRP = 8


def _sc_slots013(mtab, stab, htab, bw, wyg, midx, sidx, hidx, btn, ypg):
    info = plsc.get_sparse_core_info()
    nw = info.num_cores * info.num_subcores
    per_w = BT // nw
    n_ch = per_w // CH

    mesh = plsc.VectorSubcoreMesh(core_axis_name="c", subcore_axis_name="s")
    f32, i32 = jnp.float32, jnp.int32

    @functools.partial(
        pl.kernel,
        mesh=mesh,
        compiler_params=pltpu.CompilerParams(needs_layout_passes=False),
        out_type=jax.ShapeDtypeStruct((BT, NSLOT * D), f32),
        scratch_types=[
            pltpu.VMEM((per_w,), i32),
            pltpu.VMEM((1, per_w), i32),
            pltpu.VMEM((1, per_w), i32),
            pltpu.VMEM((3, per_w), f32),
            pltpu.VMEM((4, per_w), f32),
            pltpu.VMEM((3, D), f32),
            pltpu.VMEM((9, D), f32),
            pltpu.VMEM((3, D), f32),
            pltpu.VMEM((4, D), f32),
            pltpu.VMEM((CH, D), f32),
            pltpu.SemaphoreType.DMA,
        ],
    )
    def k(mtab_h, stab_h, htab_h, bw_h, wyg_h, midx_h, sidx_h, hidx_h,
          btn_h, ypg_h, out_h, midx_v, sidx_v, hidx_v, btn_v, ypg_v,
          st_v, ht_v, bw_v, wyg_v, rows_v, sem):
        wid = lax.axis_index("s") * info.num_cores + lax.axis_index("c")
        base = wid * per_w

        pltpu.sync_copy(midx_h.at[pl.ds(base, per_w)], midx_v)
        pltpu.sync_copy(sidx_h.at[:, pl.ds(base, per_w)], sidx_v)
        pltpu.sync_copy(hidx_h.at[:, pl.ds(base, per_w)], hidx_v)
        pltpu.sync_copy(btn_h.at[:, pl.ds(base, per_w)], btn_v)
        pltpu.sync_copy(ypg_h.at[:, pl.ds(base, per_w)], ypg_v)
        pltpu.sync_copy(stab_h, st_v)
        pltpu.sync_copy(htab_h, ht_v)
        pltpu.sync_copy(bw_h, bw_v)
        pltpu.sync_copy(wyg_h, wyg_v)

        iota16 = lax.broadcasted_iota(i32, (16,), 0)

        def splat(ref, row, t):
            return plsc.load_gather(
                ref, [jnp.full((16,), row, i32), jnp.full((16,), t, i32)])

        def chunk_body(c, carry):
            off = base + c * CH
            pltpu.async_copy(
                mtab_h.at[midx_v.at[pl.ds(c * CH, CH)]], rows_v, sem).wait()
            pltpu.sync_copy(rows_v, out_h.at[pl.ds(off, CH), pl.ds(0, D)])

            for g in range(D // (16 * GRP)):
                w = [[bw_v[kk, pl.ds((g * GRP + cc) * 16, 16)]
                      for cc in range(GRP)] for kk in range(3)]

                def s1_body(t, carry1):
                    tt = c * CH + t
                    si = splat(sidx_v, 0, tt)
                    bs = [splat(btn_v, kk, tt) for kk in range(3)]
                    for cc in range(GRP):
                        col = (g * GRP + cc) * 16
                        row = plsc.load_gather(st_v, [si, iota16 + col])
                        acc = row + bs[0] * w[0][cc] + bs[1] * w[1][cc] \
                            + bs[2] * w[2][cc]
                        rows_v[t, pl.ds(col, 16)] = acc
                    return carry1

                lax.fori_loop(0, CH, s1_body, 0)
            pltpu.sync_copy(rows_v, out_h.at[pl.ds(off, CH), pl.ds(D, D)])

            for g in range(D // (16 * GRP)):
                w = [[wyg_v[kk, pl.ds((g * GRP + cc) * 16, 16)]
                      for cc in range(GRP)] for kk in range(4)]

                def s3_body(t, carry3):
                    tt = c * CH + t
                    hi = splat(hidx_v, 0, tt)
                    ys = [splat(ypg_v, kk, tt) for kk in range(4)]
                    for cc in range(GRP):
                        col = (g * GRP + cc) * 16
                        row = plsc.load_gather(ht_v, [hi, iota16 + col])
                        acc = row + ys[0] * w[0][cc] + ys[1] * w[1][cc] \
                            + ys[2] * w[2][cc] + ys[3] * w[3][cc]
                        rows_v[t, pl.ds(col, 16)] = acc
                    return carry3

                lax.fori_loop(0, CH, s3_body, 0)
            pltpu.sync_copy(rows_v, out_h.at[pl.ds(off, CH), pl.ds(3 * D, D)])
            return carry

        lax.fori_loop(0, n_ch, chunk_body, 0)

    return k(mtab, stab, htab, bw, wyg, midx, sidx, hidx, btn, ypg)


BR = 1024


def _tc_slot2(tokens0, keys, keys_W, bias2):
    nb = BT // BR

    def body(alias_ref, keys_ref, kw_ref, bias_ref, out_ref):
        out_ref[...] = (
            jnp.dot(keys_ref[...], kw_ref[...],
                    preferred_element_type=jnp.float32)
            + bias_ref[...]
        )

    return pl.pallas_call(
        body,
        grid=(nb,),
        in_specs=[
            pl.BlockSpec(memory_space=pl.ANY),
            pl.BlockSpec((BR, 23), lambda b: (b, 0)),
            pl.BlockSpec((23, D), lambda b: (0, 0)),
            pl.BlockSpec((1, D), lambda b: (0, 0)),
        ],
        out_specs=pl.BlockSpec((BR, D), lambda b: (b, 2)),
        out_shape=jax.ShapeDtypeStruct((BT, NSLOT * D), jnp.float32),
        input_output_aliases={0: 0},
    )(tokens0, keys, keys_W, bias2)


def kernel(mouse_cat, scroll, buttons, keys, yaw_pitch, gui, hotbar,
           mouse_table, scroll_table, hotbar_table, slot_table,
           buttons_W, buttons_b, keys_W, keys_b, yawgui_W, yawgui_b):
    mtab = mouse_table + slot_table[0][None, :]
    stab = scroll_table + (slot_table[1] + buttons_b)[None, :]
    htab = hotbar_table + (slot_table[3] + yawgui_b)[None, :]
    bias2 = (slot_table[2] + keys_b)[None, :]

    midx = mouse_cat.reshape(BT).astype(jnp.int32)
    sidx = scroll.reshape(1, BT).astype(jnp.int32)
    hidx = hotbar.reshape(1, BT).astype(jnp.int32)
    btn = buttons.reshape(BT, 3).T
    ypg = jnp.concatenate([yaw_pitch, gui], axis=-1).reshape(BT, 4).T

    tokens0 = _sc_slots013(mtab, stab, htab, buttons_W, yawgui_W,
                           midx, sidx, hidx, btn, ypg)
    tokens = _tc_slot2(tokens0, keys.reshape(BT, 23), keys_W, bias2)
    return tokens.reshape(B, T, NSLOT, D)


# speedup vs baseline: 1.4677x; 1.4677x over previous
import functools

import jax
import jax.numpy as jnp
from jax import lax
from jax.experimental import pallas as pl
from jax.experimental.pallas import tpu as pltpu
from jax.experimental.pallas import tpu_sc as plsc

B, T, D = 32, 256, 1024
BT = B * T
NSLOT = 4
CH = 64
GRP = 8


def _sc_slots013(mtab, stab, htab, bw, wyg, midx, sidx, hidx, btn, ypg):
    info = plsc.get_sparse_core_info()
    nw = info.num_cores * info.num_subcores
    per_w = BT // nw
    n_ch = per_w // CH

    mesh = plsc.VectorSubcoreMesh(core_axis_name="c", subcore_axis_name="s")
    f32, i32 = jnp.float32, jnp.int32

    @functools.partial(
        pl.kernel,
        mesh=mesh,
        compiler_params=pltpu.CompilerParams(needs_layout_passes=False),
        out_type=jax.ShapeDtypeStruct((BT, NSLOT * D), f32),
        scratch_types=[
            pltpu.VMEM((per_w,), i32),
            pltpu.VMEM((1, per_w), i32),
            pltpu.VMEM((1, per_w), i32),
            pltpu.VMEM((3, per_w), f32),
            pltpu.VMEM((4, per_w), f32),
            pltpu.VMEM((3, D), f32),
            pltpu.VMEM((9, D), f32),
            pltpu.VMEM((3, D), f32),
            pltpu.VMEM((4, D), f32),
            pltpu.VMEM((CH, D), f32),
            pltpu.SemaphoreType.DMA,
        ],
    )
    def k(mtab_h, stab_h, htab_h, bw_h, wyg_h, midx_h, sidx_h, hidx_h,
          btn_h, ypg_h, out_h, midx_v, sidx_v, hidx_v, btn_v, ypg_v,
          st_v, ht_v, bw_v, wyg_v, rows_v, sem):
        wid = lax.axis_index("s") * info.num_cores + lax.axis_index("c")
        base = wid * per_w

        pltpu.sync_copy(midx_h.at[pl.ds(base, per_w)], midx_v)
        pltpu.sync_copy(sidx_h.at[:, pl.ds(base, per_w)], sidx_v)
        pltpu.sync_copy(hidx_h.at[:, pl.ds(base, per_w)], hidx_v)
        pltpu.sync_copy(btn_h.at[:, pl.ds(base, per_w)], btn_v)
        pltpu.sync_copy(ypg_h.at[:, pl.ds(base, per_w)], ypg_v)
        pltpu.sync_copy(stab_h, st_v)
        pltpu.sync_copy(htab_h, ht_v)
        pltpu.sync_copy(bw_h, bw_v)
        pltpu.sync_copy(wyg_h, wyg_v)

        iota16 = lax.broadcasted_iota(i32, (16,), 0)

        def splat(ref, row, t):
            return plsc.load_gather(
                ref, [jnp.full((16,), row, i32), jnp.full((16,), t, i32)])

        def chunk_body(c, carry):
            off = base + c * CH
            pltpu.async_copy(
                mtab_h.at[midx_v.at[pl.ds(c * CH, CH)]], rows_v, sem).wait()
            pltpu.sync_copy(rows_v, out_h.at[pl.ds(off, CH), pl.ds(0, D)])

            for g in range(D // (16 * GRP)):
                w = [[bw_v[kk, pl.ds((g * GRP + cc) * 16, 16)]
                      for cc in range(GRP)] for kk in range(3)]

                @plsc.parallel_loop(0, CH, step=1, unroll=4)
                def s1_body(t):
                    tt = c * CH + t
                    si = splat(sidx_v, 0, tt)
                    bs = [splat(btn_v, kk, tt) for kk in range(3)]
                    for cc in range(GRP):
                        col = (g * GRP + cc) * 16
                        row = plsc.load_gather(st_v, [si, iota16 + col])
                        acc = row + bs[0] * w[0][cc] + bs[1] * w[1][cc] \
                            + bs[2] * w[2][cc]
                        rows_v[t, pl.ds(col, 16)] = acc

            pltpu.sync_copy(rows_v, out_h.at[pl.ds(off, CH), pl.ds(D, D)])

            for g in range(D // (16 * GRP)):
                w = [[wyg_v[kk, pl.ds((g * GRP + cc) * 16, 16)]
                      for cc in range(GRP)] for kk in range(4)]

                @plsc.parallel_loop(0, CH, step=1, unroll=4)
                def s3_body(t):
                    tt = c * CH + t
                    hi = splat(hidx_v, 0, tt)
                    ys = [splat(ypg_v, kk, tt) for kk in range(4)]
                    for cc in range(GRP):
                        col = (g * GRP + cc) * 16
                        row = plsc.load_gather(ht_v, [hi, iota16 + col])
                        acc = row + ys[0] * w[0][cc] + ys[1] * w[1][cc] \
                            + ys[2] * w[2][cc] + ys[3] * w[3][cc]
                        rows_v[t, pl.ds(col, 16)] = acc

            pltpu.sync_copy(rows_v, out_h.at[pl.ds(off, CH), pl.ds(3 * D, D)])
            return carry

        lax.fori_loop(0, n_ch, chunk_body, 0)

    return k(mtab, stab, htab, bw, wyg, midx, sidx, hidx, btn, ypg)


BR = 1024


def _tc_slot2(tokens0, keys, keys_W, bias2):
    nb = BT // BR

    def body(alias_ref, keys_ref, kw_ref, bias_ref, out_ref):
        out_ref[...] = (
            jnp.dot(keys_ref[...], kw_ref[...],
                    preferred_element_type=jnp.float32)
            + bias_ref[...]
        )

    return pl.pallas_call(
        body,
        grid=(nb,),
        in_specs=[
            pl.BlockSpec(memory_space=pl.ANY),
            pl.BlockSpec((BR, 23), lambda b: (b, 0)),
            pl.BlockSpec((23, D), lambda b: (0, 0)),
            pl.BlockSpec((1, D), lambda b: (0, 0)),
        ],
        out_specs=pl.BlockSpec((BR, D), lambda b: (b, 2)),
        out_shape=jax.ShapeDtypeStruct((BT, NSLOT * D), jnp.float32),
        input_output_aliases={0: 0},
    )(tokens0, keys, keys_W, bias2)


def kernel(mouse_cat, scroll, buttons, keys, yaw_pitch, gui, hotbar,
           mouse_table, scroll_table, hotbar_table, slot_table,
           buttons_W, buttons_b, keys_W, keys_b, yawgui_W, yawgui_b):
    mtab = mouse_table + slot_table[0][None, :]
    stab = scroll_table + (slot_table[1] + buttons_b)[None, :]
    htab = hotbar_table + (slot_table[3] + yawgui_b)[None, :]
    bias2 = (slot_table[2] + keys_b)[None, :]

    midx = mouse_cat.reshape(BT).astype(jnp.int32)
    sidx = scroll.reshape(1, BT).astype(jnp.int32)
    hidx = hotbar.reshape(1, BT).astype(jnp.int32)
    btn = buttons.reshape(BT, 3).T
    ypg = jnp.concatenate([yaw_pitch, gui], axis=-1).reshape(BT, 4).T

    tokens0 = _sc_slots013(mtab, stab, htab, buttons_W, yawgui_W,
                           midx, sidx, hidx, btn, ypg)
    tokens = _tc_slot2(tokens0, keys.reshape(BT, 23), keys_W, bias2)
    return tokens.reshape(B, T, NSLOT, D)


# speedup vs baseline: 2.1225x; 1.4461x over previous
import functools

import jax
import jax.numpy as jnp
from jax import lax
from jax.experimental import pallas as pl
from jax.experimental.pallas import tpu as pltpu
from jax.experimental.pallas import tpu_sc as plsc

B, T, D = 32, 256, 1024
BT = B * T
NSLOT = 4


def _sc_gather_slot0(table_biased, idx_flat):
    info = plsc.get_sparse_core_info()
    nw = info.num_cores * info.num_subcores
    per_w = BT // nw
    chunk = 32
    n_chunks = per_w // chunk

    mesh = plsc.VectorSubcoreMesh(core_axis_name="c", subcore_axis_name="s")

    @functools.partial(
        pl.kernel,
        mesh=mesh,
        out_type=jax.ShapeDtypeStruct((BT, NSLOT * D), jnp.float32),
        scratch_types=[
            pltpu.VMEM((per_w,), jnp.int32),
            pltpu.VMEM((chunk, D), jnp.float32),
            pltpu.VMEM((chunk, D), jnp.float32),
            pltpu.SemaphoreType.DMA,
            pltpu.SemaphoreType.DMA,
        ],
    )
    def k(table_hbm, idx_hbm, out_hbm, idx_v, rows_a, rows_b, sem_a, sem_b):
        wid = lax.axis_index("s") * info.num_cores + lax.axis_index("c")
        base = wid * per_w
        pltpu.sync_copy(idx_hbm.at[pl.ds(base, per_w)], idx_v)

        bufs = (rows_a, rows_b)
        sems = (sem_a, sem_b)

        def gather(c):
            return pltpu.async_copy(
                table_hbm.at[idx_v.at[pl.ds(c * chunk, chunk)]],
                bufs[c % 2], sems[c % 2])

        handle = gather(0)
        for c in range(n_chunks):
            handle.wait()
            if c + 1 < n_chunks:
                handle = gather(c + 1)
            pltpu.sync_copy(
                bufs[c % 2],
                out_hbm.at[pl.ds(base + c * chunk, chunk), pl.ds(0, D)])

    return k(table_biased, idx_flat)


BR = 1024


def _tc_dense(tokens0, scroll_r, hotbar_r, buttons, keys, yaw_pitch, gui,
              scroll_table, buttons_W, keys_W, w_yp, w_gui, hotbar_table,
              bias3):
    nb = BT // BR

    def body(alias_ref, scroll_ref, hotbar_ref, btn_ref, keys_ref, yp_ref,
             gui_ref, st_ref, bw_ref, kw_ref, wyp_ref, wgui_ref, ht_ref,
             bias_ref, out_ref):
        f32 = jnp.float32
        sc = scroll_ref[0, 0, :][:, None]
        oh_s = (sc == lax.broadcasted_iota(jnp.int32, (BR, 3), 1)).astype(f32)
        out_ref[:, :D] = (
            jnp.dot(oh_s, st_ref[...], preferred_element_type=f32)
            + jnp.dot(btn_ref[...], bw_ref[...], preferred_element_type=f32)
            + bias_ref[0, 0]
        )
        out_ref[:, D:2 * D] = (
            jnp.dot(keys_ref[...], kw_ref[...], preferred_element_type=f32)
            + bias_ref[1, 0]
        )
        hb = hotbar_ref[0, 0, :][:, None]
        oh_h = (hb == lax.broadcasted_iota(jnp.int32, (BR, 9), 1)).astype(f32)
        out_ref[:, 2 * D:] = (
            jnp.dot(yp_ref[...], wyp_ref[...], preferred_element_type=f32)
            + jnp.dot(gui_ref[...], wgui_ref[...], preferred_element_type=f32)
            + jnp.dot(oh_h, ht_ref[...], preferred_element_type=f32)
            + bias_ref[2, 0]
        )

    full = lambda shape: pl.BlockSpec(shape, lambda b: (0,) * len(shape))
    per_b = lambda shape: pl.BlockSpec(shape, lambda b: (b,) + (0,) * (len(shape) - 1))

    return pl.pallas_call(
        body,
        grid=(nb,),
        in_specs=[
            pl.BlockSpec(memory_space=pl.ANY),
            per_b((1, 1, BR)),
            per_b((1, 1, BR)),
            per_b((BR, 3)),
            per_b((BR, 23)),
            per_b((BR, 2)),
            per_b((BR, 2)),
            full((3, D)),
            full((3, D)),
            full((23, D)),
            full((2, D)),
            full((2, D)),
            full((9, D)),
            full((3, 1, D)),
        ],
        out_specs=pl.BlockSpec((pl.Element(BR), pl.Element(3 * D)),
                               lambda b: (b * BR, D)),
        out_shape=jax.ShapeDtypeStruct((BT, NSLOT * D), jnp.float32),
        input_output_aliases={0: 0},
    )(tokens0, scroll_r, hotbar_r, buttons, keys, yaw_pitch, gui,
      scroll_table, buttons_W, keys_W, w_yp, w_gui, hotbar_table, bias3)


def kernel(mouse_cat, scroll, buttons, keys, yaw_pitch, gui, hotbar,
           mouse_table, scroll_table, hotbar_table, slot_table,
           buttons_W, buttons_b, keys_W, keys_b, yawgui_W, yawgui_b):
    table_biased = mouse_table + slot_table[0][None, :]
    bias3 = jnp.stack([
        slot_table[1] + buttons_b,
        slot_table[2] + keys_b,
        slot_table[3] + yawgui_b,
    ])[:, None, :]
    w_yp = yawgui_W[:2]
    w_gui = yawgui_W[2:]

    idx_flat = mouse_cat.reshape(BT).astype(jnp.int32)
    scroll_r = scroll.reshape(BT // BR, 1, BR).astype(jnp.int32)
    hotbar_r = hotbar.reshape(BT // BR, 1, BR).astype(jnp.int32)

    tokens0 = _sc_gather_slot0(table_biased, idx_flat)
    tokens = _tc_dense(tokens0, scroll_r, hotbar_r,
                       buttons.reshape(BT, 3), keys.reshape(BT, 23),
                       yaw_pitch.reshape(BT, 2), gui.reshape(BT, 2),
                       scroll_table, buttons_W, keys_W,
                       w_yp, w_gui, hotbar_table, bias3)
    return tokens.reshape(B, T, NSLOT, D)
